# baseline (speedup 1.0000x reference)
import jax
import jax.numpy as jnp
from jax.experimental import pallas as pl
from jax.experimental.pallas import tpu as pltpu

N = 10000
E = 320000
F = 128
H = 256
A = 17
TW = 144
EB = 512
NB = 1000

_bf16 = jnp.bfloat16
_f32 = jnp.float32


def _edge_body(gs, gt, ea, ws, wt, wa, be, wc1, bc1, wc2, ef_out, tr_out):
    g1 = gs[:, :F].astype(_bf16)
    g2 = gt[:, :F].astype(_bf16)
    c1 = gs[:, F:TW]
    c2 = gt[:, F:TW]
    x = jnp.dot(g1, ws[...], preferred_element_type=_f32)
    x = x + jnp.dot(g2, wt[...], preferred_element_type=_f32)
    x = x + jnp.dot(ea[...].astype(_bf16), wa[...], preferred_element_type=_f32)
    x = x + be[...]
    ef = jnp.maximum(x, 0.0)
    ef_out[...] = ef
    h = jnp.dot(ef.astype(_bf16), wc1[...], preferred_element_type=_f32) + bc1[...]
    h = jnp.maximum(h, 0.0)
    phi = jnp.sum(h * wc2[...], axis=1, keepdims=True)
    tr_out[...] = (c1 - c2) * phi


def _edge_mlp(G, edge_attr, ws, wt, wa, be2, wc1b, bc12, wc22):
    nsteps = E // EB
    return pl.pallas_call(
        _edge_body,
        grid=(nsteps,),
        in_specs=[
            pl.BlockSpec((EB, TW), lambda i: (i, 0)),
            pl.BlockSpec((EB, TW), lambda i, _n=nsteps: (i + _n, 0)),
            pl.BlockSpec((EB, A), lambda i: (i, 0)),
            pl.BlockSpec((F, H), lambda i: (0, 0)),
            pl.BlockSpec((F, H), lambda i: (0, 0)),
            pl.BlockSpec((A, H), lambda i: (0, 0)),
            pl.BlockSpec((1, H), lambda i: (0, 0)),
            pl.BlockSpec((H, H), lambda i: (0, 0)),
            pl.BlockSpec((1, H), lambda i: (0, 0)),
            pl.BlockSpec((1, H), lambda i: (0, 0)),
        ],
        out_specs=[
            pl.BlockSpec((EB, H), lambda i: (i, 0)),
            pl.BlockSpec((EB, 16), lambda i: (i, 0)),
        ],
        out_shape=[
            jax.ShapeDtypeStruct((E, H), _f32),
            jax.ShapeDtypeStruct((E, 16), _f32),
        ],
    )(G, G, edge_attr, ws, wt, wa, be2, wc1b, bc12, wc22)


def _node_body(nf, agg, co, cs, wna, wnb, bn, no_out, c_out):
    x = jnp.dot(nf[...].astype(_bf16), wna[...], preferred_element_type=_f32)
    x = x + jnp.dot(agg[...].astype(_bf16), wnb[...], preferred_element_type=_f32)
    x = x + bn[...]
    no_out[...] = jnp.maximum(x, 0.0)
    c_out[...] = co[...] + cs[...]


def _node_mlp(node_feats, agg, co16, cs, wna, wnb, bn2):
    return pl.pallas_call(
        _node_body,
        grid=(N // NB,),
        in_specs=[
            pl.BlockSpec((NB, F), lambda i: (i, 0)),
            pl.BlockSpec((NB, H), lambda i: (i, 0)),
            pl.BlockSpec((NB, 16), lambda i: (i, 0)),
            pl.BlockSpec((NB, 16), lambda i: (i, 0)),
            pl.BlockSpec((F, H), lambda i: (0, 0)),
            pl.BlockSpec((H, H), lambda i: (0, 0)),
            pl.BlockSpec((1, H), lambda i: (0, 0)),
        ],
        out_specs=[
            pl.BlockSpec((NB, H), lambda i: (i, 0)),
            pl.BlockSpec((NB, 16), lambda i: (i, 0)),
        ],
        out_shape=[
            jax.ShapeDtypeStruct((N, H), _f32),
            jax.ShapeDtypeStruct((N, 16), _f32),
        ],
    )(node_feats, agg, co16, cs, wna, wnb, bn2)


def kernel(node_feats, edge_index, edge_attr, coords, We, be, Wn, bn, Wc1, bc1, Wc2):
    co16 = jnp.pad(coords, ((0, 0), (0, 13)))
    T = jnp.concatenate([node_feats, co16], axis=1)
    rc = edge_index.reshape(2 * E)

    G = jnp.take(T, rc, axis=0)

    ws = We[:, :F].T.astype(_bf16)
    wt = We[:, F:2 * F].T.astype(_bf16)
    wa = We[:, 2 * F:].T.astype(_bf16)
    be2 = be.reshape(1, H)
    wc1b = Wc1.T.astype(_bf16)
    bc12 = bc1.reshape(1, H)
    wc22 = Wc2.reshape(1, H)

    ef, tr = _edge_mlp(G, edge_attr, ws, wt, wa, be2, wc1b, bc12, wc22)

    row = edge_index[0]
    agg = jax.ops.segment_sum(ef, row, num_segments=N)
    cs = jax.ops.segment_sum(tr, row, num_segments=N)

    wna = Wn[:, :F].T.astype(_bf16)
    wnb = Wn[:, F:].T.astype(_bf16)
    bn2 = bn.reshape(1, H)
    node_out, c_out = _node_mlp(node_feats, agg, co16, cs, wna, wnb, bn2)
    return node_out, ef, c_out[:, :3]


# speedup vs baseline: 1.3458x; 1.3458x over previous
import jax
import jax.numpy as jnp
from jax.experimental import pallas as pl
from jax.experimental.pallas import tpu as pltpu
from jax.experimental.pallas import tpu_sc as plsc

N = 10000
E = 320000
F = 128
H = 256
A = 17
TW = 144
EB = 512
NB = 1000

_bf16 = jnp.bfloat16
_f32 = jnp.float32


GW = 128


def _sc_gather(T, rc2d):
    mesh = plsc.VectorSubcoreMesh(core_axis_name="c", subcore_axis_name="s")

    @pl.kernel(out_type=jax.ShapeDtypeStruct((2 * E, F), _f32), mesh=mesh)
    def k(t_hbm, i_hbm, o_hbm):
        def body(i_vmem, o_vmem):
            pltpu.sync_copy(t_hbm.at[i_vmem.at[0]], o_vmem)

        pltpu.emit_pipeline(
            body,
            grid=(2 * E // GW,),
            in_specs=[pl.BlockSpec((1, GW), lambda i: (0, i))],
            out_specs=[pl.BlockSpec((GW, F), lambda i: (i, 0))],
            core_axis_name=("c", "s"),
            dimension_semantics=(pltpu.PARALLEL,),
        )(i_hbm, o_hbm)

    return k(T, rc2d)


def _edge_body(gs, gt, ea, ws, wt, wa, be, wc1, bc1, wc2, ef_out, phi_out):
    g1 = gs[...].astype(_bf16)
    g2 = gt[...].astype(_bf16)
    x = jnp.dot(g1, ws[...], preferred_element_type=_f32)
    x = x + jnp.dot(g2, wt[...], preferred_element_type=_f32)
    x = x + jnp.dot(ea[...].astype(_bf16), wa[...], preferred_element_type=_f32)
    x = x + be[...]
    ef = jnp.maximum(x, 0.0)
    ef_out[...] = ef
    h = jnp.dot(ef.astype(_bf16), wc1[...], preferred_element_type=_f32) + bc1[...]
    h = jnp.maximum(h, 0.0)
    phi = jnp.sum(h * wc2[...], axis=1)
    phi_out[...] = phi.reshape(1, EB // 128, 128)


def _edge_mlp(G, edge_attr, ws, wt, wa, be2, wc1b, bc12, wc22):
    nsteps = E // EB
    return pl.pallas_call(
        _edge_body,
        grid=(nsteps,),
        in_specs=[
            pl.BlockSpec((EB, F), lambda i: (i, 0)),
            pl.BlockSpec((EB, F), lambda i, _n=nsteps: (i + _n, 0)),
            pl.BlockSpec((EB, A), lambda i: (i, 0)),
            pl.BlockSpec((F, H), lambda i: (0, 0)),
            pl.BlockSpec((F, H), lambda i: (0, 0)),
            pl.BlockSpec((A, H), lambda i: (0, 0)),
            pl.BlockSpec((1, H), lambda i: (0, 0)),
            pl.BlockSpec((H, H), lambda i: (0, 0)),
            pl.BlockSpec((1, H), lambda i: (0, 0)),
            pl.BlockSpec((1, H), lambda i: (0, 0)),
        ],
        out_specs=[
            pl.BlockSpec((EB, H), lambda i: (i, 0)),
            pl.BlockSpec((1, EB // 128, 128), lambda i: (i, 0, 0)),
        ],
        out_shape=[
            jax.ShapeDtypeStruct((E, H), _f32),
            jax.ShapeDtypeStruct((E // EB, EB // 128, 128), _f32),
        ],
    )(G, G, edge_attr, ws, wt, wa, be2, wc1b, bc12, wc22)


def _node_body(nf, agg, co, cs, wna, wnb, bn, no_out, c_out):
    x = jnp.dot(nf[...].astype(_bf16), wna[...], preferred_element_type=_f32)
    x = x + jnp.dot(agg[...].astype(_bf16), wnb[...], preferred_element_type=_f32)
    x = x + bn[...]
    no_out[...] = jnp.maximum(x, 0.0)
    c_out[...] = co[...] + cs[...]


def _node_mlp(node_feats, agg, co16, cs, wna, wnb, bn2):
    return pl.pallas_call(
        _node_body,
        grid=(N // NB,),
        in_specs=[
            pl.BlockSpec((NB, F), lambda i: (i, 0)),
            pl.BlockSpec((NB, H), lambda i: (i, 0)),
            pl.BlockSpec((NB, 16), lambda i: (i, 0)),
            pl.BlockSpec((NB, 16), lambda i: (i, 0)),
            pl.BlockSpec((F, H), lambda i: (0, 0)),
            pl.BlockSpec((H, H), lambda i: (0, 0)),
            pl.BlockSpec((1, H), lambda i: (0, 0)),
        ],
        out_specs=[
            pl.BlockSpec((NB, H), lambda i: (i, 0)),
            pl.BlockSpec((NB, 16), lambda i: (i, 0)),
        ],
        out_shape=[
            jax.ShapeDtypeStruct((N, H), _f32),
            jax.ShapeDtypeStruct((N, 16), _f32),
        ],
    )(node_feats, agg, co16, cs, wna, wnb, bn2)


def kernel(node_feats, edge_index, edge_attr, coords, We, be, Wn, bn, Wc1, bc1, Wc2):
    co16 = jnp.pad(coords, ((0, 0), (0, 13)))
    rc2d = edge_index.reshape(1, 2 * E)

    G = _sc_gather(node_feats, rc2d)

    ws = We[:, :F].T.astype(_bf16)
    wt = We[:, F:2 * F].T.astype(_bf16)
    wa = We[:, 2 * F:].T.astype(_bf16)
    be2 = be.reshape(1, H)
    wc1b = Wc1.T.astype(_bf16)
    bc12 = bc1.reshape(1, H)
    wc22 = Wc2.reshape(1, H)

    ef, phi_pack = _edge_mlp(G, edge_attr, ws, wt, wa, be2, wc1b, bc12, wc22)

    row = edge_index[0]
    col = edge_index[1]
    phi = phi_pack.reshape(E, 1)
    tr = (co16[row] - co16[col]) * phi
    agg = jax.ops.segment_sum(ef, row, num_segments=N)
    cs = jax.ops.segment_sum(tr, row, num_segments=N)

    wna = Wn[:, :F].T.astype(_bf16)
    wnb = Wn[:, F:].T.astype(_bf16)
    bn2 = bn.reshape(1, H)
    node_out, c_out = _node_mlp(node_feats, agg, co16, cs, wna, wnb, bn2)
    return node_out, ef, c_out[:, :3]


# speedup vs baseline: 4.5896x; 3.4103x over previous
import dataclasses

import jax
import jax.numpy as jnp
from jax import lax
from jax.experimental import pallas as pl
from jax.experimental.pallas import tpu as pltpu
from jax.experimental.pallas import tpu_sc as plsc

N = 10000
E = 320000
F = 128
H = 256
A = 17
TW = 144
EB = 512
NB = 1000

_bf16 = jnp.bfloat16
_f32 = jnp.float32


GW = 128


def _sc_gather(T, rc2d):
    mesh = plsc.VectorSubcoreMesh(core_axis_name="c", subcore_axis_name="s")

    @pl.kernel(out_type=jax.ShapeDtypeStruct((2 * E, F), _f32), mesh=mesh)
    def k(t_hbm, i_hbm, o_hbm):
        def body(i_vmem, o_vmem):
            pltpu.sync_copy(t_hbm.at[i_vmem.at[0]], o_vmem)

        pltpu.emit_pipeline(
            body,
            grid=(2 * E // GW,),
            in_specs=[pl.BlockSpec((1, GW), lambda i: (0, i))],
            out_specs=[pl.BlockSpec((GW, F), lambda i: (i, 0))],
            core_axis_name=("c", "s"),
            dimension_semantics=(pltpu.PARALLEL,),
        )(i_hbm, o_hbm)

    return k(T, rc2d)


EBC = 128
NT = N // 16


NW = 624


def _sc_agg(ef, rc2d, z128):
    mesh = plsc.VectorSubcoreMesh(core_axis_name="c", subcore_axis_name="s")

    @pl.kernel(
        out_type=jax.ShapeDtypeStruct((N, H), _f32),
        mesh=mesh,
        scratch_types=[pltpu.VMEM_SHARED((N, F), _f32)],
    )
    def k(ef_hbm, rc_hbm, z128_hbm, agg_hbm, agg_sh):
        cid = lax.axis_index("c")
        sid = lax.axis_index("s")
        pltpu.sync_copy(z128_hbm, agg_sh.at[pl.ds(sid * NT, NT)])
        plsc.subcore_barrier()

        def body(ef_vmem, ridx_vmem):
            pltpu.sync_copy(ef_vmem, agg_sh.at[ridx_vmem.at[0]], add=True)

        pltpu.emit_pipeline(
            body,
            grid=(E // EBC,),
            in_specs=[
                pl.BlockSpec((EBC, F), lambda i: (i, cid)),
                pl.BlockSpec((1, EBC), lambda i: (0, i)),
            ],
            core_axis_name="s",
            dimension_semantics=(pltpu.PARALLEL,),
        )(ef_hbm, rc_hbm)

        plsc.subcore_barrier()
        pltpu.sync_copy(agg_sh.at[pl.ds(sid * NW, NW)],
                        agg_hbm.at[pl.ds(sid * NW, NW), pl.ds(cid * F, F)])

        @pl.when(sid == 15)
        def _tail():
            pltpu.sync_copy(agg_sh.at[pl.ds(16 * NW, N - 16 * NW)],
                            agg_hbm.at[pl.ds(16 * NW, N - 16 * NW), pl.ds(cid * F, F)])

    return k(ef, rc2d, z128)


def _sc_coords(phi_pack, rc2d, coords3, z128):
    mesh = plsc.VectorSubcoreMesh(core_axis_name="c", subcore_axis_name="s")

    cp = pltpu.CompilerParams()
    if "needs_layout_passes" in pltpu.CompilerParams.__dataclass_fields__:
        cp = dataclasses.replace(cp, needs_layout_passes=False)

    @pl.kernel(
        out_type=jax.ShapeDtypeStruct((2, N, 128), _f32),
        mesh=mesh,
        compiler_params=cp,
        scratch_types=[
            pltpu.VMEM_SHARED((N, 128), _f32),
            pltpu.VMEM((3, N), _f32),
            pltpu.VMEM((64, 128), _f32),
            pltpu.VMEM((1, 64), jnp.int32),
        ],
    )
    def k(phi_hbm, rc_hbm, c3_hbm, z128_hbm, csp_hbm, cs_sh, ctab, stage, idx_s):
        cid = lax.axis_index("c")
        sid = lax.axis_index("s")
        pltpu.sync_copy(c3_hbm, ctab)
        pltpu.sync_copy(z128_hbm, cs_sh.at[pl.ds(sid * NT, NT)])
        pltpu.sync_copy(z128_hbm.at[pl.ds(0, 64)], stage)
        plsc.subcore_barrier()

        lanes = lax.iota(jnp.int32, 16)

        def body(ridx_vmem, cidx_vmem, phi_vmem):
            for j in range(4):
                sl = pl.ds(cid * 64 + j * 16, 16)
                idxr = ridx_vmem[0, sl]
                idxc = cidx_vmem[0, sl]
                p = phi_vmem[0, 0, sl]
                idx_s[0, pl.ds(j * 16, 16)] = idxr
                rows = jnp.full((16,), j * 16, jnp.int32) + lanes
                for d in range(3):
                    dsplat = jnp.full((16,), d, jnp.int32)
                    a = plsc.load_gather(ctab, [dsplat, idxr])
                    b = plsc.load_gather(ctab, [dsplat, idxc])
                    plsc.store_scatter(stage, [rows, dsplat], (a - b) * p)
            pltpu.sync_copy(stage, cs_sh.at[idx_s.at[0]], add=True)

        pltpu.emit_pipeline(
            body,
            grid=(E // EBC,),
            in_specs=[
                pl.BlockSpec((1, EBC), lambda i: (0, i)),
                pl.BlockSpec((1, EBC), lambda i: (0, E // EBC + i)),
                pl.BlockSpec((1, 1, 128), lambda i: (i // 4, i % 4, 0)),
            ],
            core_axis_name="s",
            dimension_semantics=(pltpu.PARALLEL,),
        )(rc_hbm, rc_hbm, phi_hbm)

        plsc.subcore_barrier()
        pltpu.sync_copy(cs_sh.at[pl.ds(sid * NW, NW)],
                        csp_hbm.at[cid].at[pl.ds(sid * NW, NW)])

        @pl.when(sid == 15)
        def _tail():
            pltpu.sync_copy(cs_sh.at[pl.ds(16 * NW, N - 16 * NW)],
                            csp_hbm.at[cid].at[pl.ds(16 * NW, N - 16 * NW)])

    return k(phi_pack, rc2d, coords3, z128)


def _edge_body(gs, gt, ea, ws, wt, wa, be, wc1, bc1, wc2, ef_out, phi_out):
    g1 = gs[...].astype(_bf16)
    g2 = gt[...].astype(_bf16)
    x = jnp.dot(g1, ws[...], preferred_element_type=_f32)
    x = x + jnp.dot(g2, wt[...], preferred_element_type=_f32)
    x = x + jnp.dot(ea[...].astype(_bf16), wa[...], preferred_element_type=_f32)
    x = x + be[...]
    ef = jnp.maximum(x, 0.0)
    ef_out[...] = ef
    h = jnp.dot(ef.astype(_bf16), wc1[...], preferred_element_type=_f32) + bc1[...]
    h = jnp.maximum(h, 0.0)
    phi = jnp.sum(h * wc2[...], axis=1)
    phi_out[...] = phi.reshape(1, EB // 128, 128)


def _edge_mlp(G, edge_attr, ws, wt, wa, be2, wc1b, bc12, wc22):
    nsteps = E // EB
    return pl.pallas_call(
        _edge_body,
        grid=(nsteps,),
        in_specs=[
            pl.BlockSpec((EB, F), lambda i: (i, 0)),
            pl.BlockSpec((EB, F), lambda i, _n=nsteps: (i + _n, 0)),
            pl.BlockSpec((EB, A), lambda i: (i, 0)),
            pl.BlockSpec((F, H), lambda i: (0, 0)),
            pl.BlockSpec((F, H), lambda i: (0, 0)),
            pl.BlockSpec((A, H), lambda i: (0, 0)),
            pl.BlockSpec((1, H), lambda i: (0, 0)),
            pl.BlockSpec((H, H), lambda i: (0, 0)),
            pl.BlockSpec((1, H), lambda i: (0, 0)),
            pl.BlockSpec((1, H), lambda i: (0, 0)),
        ],
        out_specs=[
            pl.BlockSpec((EB, H), lambda i: (i, 0)),
            pl.BlockSpec((1, EB // 128, 128), lambda i: (i, 0, 0)),
        ],
        out_shape=[
            jax.ShapeDtypeStruct((E, H), _f32),
            jax.ShapeDtypeStruct((E // EB, EB // 128, 128), _f32),
        ],
    )(G, G, edge_attr, ws, wt, wa, be2, wc1b, bc12, wc22)


def _node_body(nf, agg, co, cs0, cs1, wna, wnb, bn, no_out, c_out):
    x = jnp.dot(nf[...].astype(_bf16), wna[...], preferred_element_type=_f32)
    x = x + jnp.dot(agg[...].astype(_bf16), wnb[...], preferred_element_type=_f32)
    x = x + bn[...]
    no_out[...] = jnp.maximum(x, 0.0)
    c_out[...] = co[...] + cs0[0][:, :16] + cs1[0][:, :16]


def _node_mlp(node_feats, agg, co16, csp, wna, wnb, bn2):
    return pl.pallas_call(
        _node_body,
        grid=(N // NB,),
        in_specs=[
            pl.BlockSpec((NB, F), lambda i: (i, 0)),
            pl.BlockSpec((NB, H), lambda i: (i, 0)),
            pl.BlockSpec((NB, 16), lambda i: (i, 0)),
            pl.BlockSpec((1, NB, 128), lambda i: (0, i, 0)),
            pl.BlockSpec((1, NB, 128), lambda i: (1, i, 0)),
            pl.BlockSpec((F, H), lambda i: (0, 0)),
            pl.BlockSpec((H, H), lambda i: (0, 0)),
            pl.BlockSpec((1, H), lambda i: (0, 0)),
        ],
        out_specs=[
            pl.BlockSpec((NB, H), lambda i: (i, 0)),
            pl.BlockSpec((NB, 16), lambda i: (i, 0)),
        ],
        out_shape=[
            jax.ShapeDtypeStruct((N, H), _f32),
            jax.ShapeDtypeStruct((N, 16), _f32),
        ],
    )(node_feats, agg, co16, csp, csp, wna, wnb, bn2)


def kernel(node_feats, edge_index, edge_attr, coords, We, be, Wn, bn, Wc1, bc1, Wc2):
    co16 = jnp.pad(coords, ((0, 0), (0, 13)))
    rc2d = edge_index.reshape(1, 2 * E)

    G = _sc_gather(node_feats, rc2d)

    ws = We[:, :F].T.astype(_bf16)
    wt = We[:, F:2 * F].T.astype(_bf16)
    wa = We[:, 2 * F:].T.astype(_bf16)
    be2 = be.reshape(1, H)
    wc1b = Wc1.T.astype(_bf16)
    bc12 = bc1.reshape(1, H)
    wc22 = Wc2.reshape(1, H)

    ef, phi_pack = _edge_mlp(G, edge_attr, ws, wt, wa, be2, wc1b, bc12, wc22)

    coords3 = coords.T
    z128 = jnp.zeros((NT, F), _f32)
    agg = _sc_agg(ef, rc2d, z128)
    csp = _sc_coords(phi_pack, rc2d, coords3, z128)

    wna = Wn[:, :F].T.astype(_bf16)
    wnb = Wn[:, F:].T.astype(_bf16)
    bn2 = bn.reshape(1, H)
    node_out, c_out = _node_mlp(node_feats, agg, co16, csp, wna, wnb, bn2)
    return node_out, ef, c_out[:, :3]


# speedup vs baseline: 4.6674x; 1.0169x over previous
import dataclasses

import jax
import jax.numpy as jnp
from jax import lax
from jax.experimental import pallas as pl
from jax.experimental.pallas import tpu as pltpu
from jax.experimental.pallas import tpu_sc as plsc

N = 10000
E = 320000
F = 128
H = 256
A = 17
TW = 144
EB = 512
NB = 1000

_bf16 = jnp.bfloat16
_f32 = jnp.float32


GW = 128


def _sc_gather(T, rc2d):
    mesh = plsc.VectorSubcoreMesh(core_axis_name="c", subcore_axis_name="s")

    @pl.kernel(out_type=jax.ShapeDtypeStruct((E, 2 * F), _f32), mesh=mesh)
    def k(t_hbm, i_hbm, o_hbm):
        def body(ir_vmem, ic_vmem, o1_vmem, o2_vmem):
            pltpu.sync_copy(t_hbm.at[ir_vmem.at[0]], o1_vmem)
            pltpu.sync_copy(t_hbm.at[ic_vmem.at[0]], o2_vmem)

        pltpu.emit_pipeline(
            body,
            grid=(E // GW,),
            in_specs=[pl.BlockSpec((1, GW), lambda i: (0, i)),
                      pl.BlockSpec((1, GW), lambda i: (0, E // GW + i))],
            out_specs=[pl.BlockSpec((GW, F), lambda i: (i, 0)),
                       pl.BlockSpec((GW, F), lambda i: (i, 1))],
            core_axis_name=("c", "s"),
            dimension_semantics=(pltpu.PARALLEL,),
        )(i_hbm, i_hbm, o_hbm, o_hbm)

    return k(T, rc2d)


EBC = 128
NT = N // 16


NW = 624


def _sc_agg(ef, rc2d, z128):
    mesh = plsc.VectorSubcoreMesh(core_axis_name="c", subcore_axis_name="s")

    @pl.kernel(
        out_type=jax.ShapeDtypeStruct((N, H), _f32),
        mesh=mesh,
        scratch_types=[pltpu.VMEM_SHARED((N, F), _f32)],
    )
    def k(ef_hbm, rc_hbm, z128_hbm, agg_hbm, agg_sh):
        cid = lax.axis_index("c")
        sid = lax.axis_index("s")
        pltpu.sync_copy(z128_hbm, agg_sh.at[pl.ds(sid * NT, NT)])
        plsc.subcore_barrier()

        def body(ef_vmem, ridx_vmem):
            pltpu.sync_copy(ef_vmem, agg_sh.at[ridx_vmem.at[0]], add=True)

        pltpu.emit_pipeline(
            body,
            grid=(E // EBC,),
            in_specs=[
                pl.BlockSpec((EBC, F), lambda i: (i, cid)),
                pl.BlockSpec((1, EBC), lambda i: (0, i)),
            ],
            core_axis_name="s",
            dimension_semantics=(pltpu.PARALLEL,),
        )(ef_hbm, rc_hbm)

        plsc.subcore_barrier()
        pltpu.sync_copy(agg_sh.at[pl.ds(sid * NW, NW)],
                        agg_hbm.at[pl.ds(sid * NW, NW), pl.ds(cid * F, F)])

        @pl.when(sid == 15)
        def _tail():
            pltpu.sync_copy(agg_sh.at[pl.ds(16 * NW, N - 16 * NW)],
                            agg_hbm.at[pl.ds(16 * NW, N - 16 * NW), pl.ds(cid * F, F)])

    return k(ef, rc2d, z128)


def _sc_coords(phi_pack, rc2d, coords3, z128):
    mesh = plsc.VectorSubcoreMesh(core_axis_name="c", subcore_axis_name="s")

    cp = pltpu.CompilerParams()
    if "needs_layout_passes" in pltpu.CompilerParams.__dataclass_fields__:
        cp = dataclasses.replace(cp, needs_layout_passes=False)

    @pl.kernel(
        out_type=jax.ShapeDtypeStruct((2, N, 128), _f32),
        mesh=mesh,
        compiler_params=cp,
        scratch_types=[
            pltpu.VMEM_SHARED((N, 128), _f32),
            pltpu.VMEM((3, N), _f32),
            pltpu.VMEM((64, 128), _f32),
            pltpu.VMEM((1, 64), jnp.int32),
        ],
    )
    def k(phi_hbm, rc_hbm, c3_hbm, z128_hbm, csp_hbm, cs_sh, ctab, stage, idx_s):
        cid = lax.axis_index("c")
        sid = lax.axis_index("s")
        pltpu.sync_copy(c3_hbm, ctab)
        pltpu.sync_copy(z128_hbm, cs_sh.at[pl.ds(sid * NT, NT)])
        pltpu.sync_copy(z128_hbm.at[pl.ds(0, 64)], stage)
        plsc.subcore_barrier()

        lanes = lax.iota(jnp.int32, 16)

        def body(ridx_vmem, cidx_vmem, phi_vmem):
            for j in range(4):
                sl = pl.ds(cid * 64 + j * 16, 16)
                idxr = ridx_vmem[0, sl]
                idxc = cidx_vmem[0, sl]
                p = phi_vmem[0, 0, sl]
                idx_s[0, pl.ds(j * 16, 16)] = idxr
                rows = jnp.full((16,), j * 16, jnp.int32) + lanes
                for d in range(3):
                    dsplat = jnp.full((16,), d, jnp.int32)
                    a = plsc.load_gather(ctab, [dsplat, idxr])
                    b = plsc.load_gather(ctab, [dsplat, idxc])
                    plsc.store_scatter(stage, [rows, dsplat], (a - b) * p)
            pltpu.sync_copy(stage, cs_sh.at[idx_s.at[0]], add=True)

        pltpu.emit_pipeline(
            body,
            grid=(E // EBC,),
            in_specs=[
                pl.BlockSpec((1, EBC), lambda i: (0, i)),
                pl.BlockSpec((1, EBC), lambda i: (0, E // EBC + i)),
                pl.BlockSpec((1, 1, 128), lambda i: (i // 4, i % 4, 0)),
            ],
            core_axis_name="s",
            dimension_semantics=(pltpu.PARALLEL,),
        )(rc_hbm, rc_hbm, phi_hbm)

        plsc.subcore_barrier()
        pltpu.sync_copy(cs_sh.at[pl.ds(sid * NW, NW)],
                        csp_hbm.at[cid].at[pl.ds(sid * NW, NW)])

        @pl.when(sid == 15)
        def _tail():
            pltpu.sync_copy(cs_sh.at[pl.ds(16 * NW, N - 16 * NW)],
                            csp_hbm.at[cid].at[pl.ds(16 * NW, N - 16 * NW)])

    return k(phi_pack, rc2d, coords3, z128)


def _edge_body(gs, ea, ws, wa, be, wc1, bc1, wc2, ef_out, phi_out):
    g = gs[...].astype(_bf16)
    x = jnp.dot(g, ws[...], preferred_element_type=_f32)
    x = x + jnp.dot(ea[...].astype(_bf16), wa[...], preferred_element_type=_f32)
    x = x + be[...]
    ef = jnp.maximum(x, 0.0)
    ef_out[...] = ef
    h = jnp.dot(ef.astype(_bf16), wc1[...], preferred_element_type=_f32) + bc1[...]
    h = jnp.maximum(h, 0.0)
    phi = jnp.dot(h.astype(_bf16), wc2[...], preferred_element_type=_f32)
    phi_out[...] = phi.reshape(1, EB // 128, 128)


def _edge_mlp(G, edge_attr, ws, wa, be2, wc1b, bc12, wc22):
    nsteps = E // EB
    return pl.pallas_call(
        _edge_body,
        grid=(nsteps,),
        in_specs=[
            pl.BlockSpec((EB, 2 * F), lambda i: (i, 0)),
            pl.BlockSpec((EB, A), lambda i: (i, 0)),
            pl.BlockSpec((2 * F, H), lambda i: (0, 0)),
            pl.BlockSpec((A, H), lambda i: (0, 0)),
            pl.BlockSpec((1, H), lambda i: (0, 0)),
            pl.BlockSpec((H, H), lambda i: (0, 0)),
            pl.BlockSpec((1, H), lambda i: (0, 0)),
            pl.BlockSpec((H, 1), lambda i: (0, 0)),
        ],
        out_specs=[
            pl.BlockSpec((EB, H), lambda i: (i, 0)),
            pl.BlockSpec((1, EB // 128, 128), lambda i: (i, 0, 0)),
        ],
        out_shape=[
            jax.ShapeDtypeStruct((E, H), _f32),
            jax.ShapeDtypeStruct((E // EB, EB // 128, 128), _f32),
        ],
    )(G, edge_attr, ws, wa, be2, wc1b, bc12, wc22)


def _node_body(nf, agg, co, cs0, cs1, wna, wnb, bn, no_out, c_out):
    x = jnp.dot(nf[...].astype(_bf16), wna[...], preferred_element_type=_f32)
    x = x + jnp.dot(agg[...].astype(_bf16), wnb[...], preferred_element_type=_f32)
    x = x + bn[...]
    no_out[...] = jnp.maximum(x, 0.0)
    c_out[...] = co[...] + cs0[0][:, :16] + cs1[0][:, :16]


def _node_mlp(node_feats, agg, co16, csp, wna, wnb, bn2):
    return pl.pallas_call(
        _node_body,
        grid=(N // NB,),
        in_specs=[
            pl.BlockSpec((NB, F), lambda i: (i, 0)),
            pl.BlockSpec((NB, H), lambda i: (i, 0)),
            pl.BlockSpec((NB, 16), lambda i: (i, 0)),
            pl.BlockSpec((1, NB, 128), lambda i: (0, i, 0)),
            pl.BlockSpec((1, NB, 128), lambda i: (1, i, 0)),
            pl.BlockSpec((F, H), lambda i: (0, 0)),
            pl.BlockSpec((H, H), lambda i: (0, 0)),
            pl.BlockSpec((1, H), lambda i: (0, 0)),
        ],
        out_specs=[
            pl.BlockSpec((NB, H), lambda i: (i, 0)),
            pl.BlockSpec((NB, 16), lambda i: (i, 0)),
        ],
        out_shape=[
            jax.ShapeDtypeStruct((N, H), _f32),
            jax.ShapeDtypeStruct((N, 16), _f32),
        ],
    )(node_feats, agg, co16, csp, csp, wna, wnb, bn2)


def kernel(node_feats, edge_index, edge_attr, coords, We, be, Wn, bn, Wc1, bc1, Wc2):
    co16 = jnp.pad(coords, ((0, 0), (0, 13)))
    rc2d = edge_index.reshape(1, 2 * E)

    G = _sc_gather(node_feats, rc2d)

    ws = We[:, :2 * F].T.astype(_bf16)
    wa = We[:, 2 * F:].T.astype(_bf16)
    be2 = be.reshape(1, H)
    wc1b = Wc1.T.astype(_bf16)
    bc12 = bc1.reshape(1, H)
    wc22 = Wc2.reshape(H, 1).astype(_bf16)

    ef, phi_pack = _edge_mlp(G, edge_attr, ws, wa, be2, wc1b, bc12, wc22)

    coords3 = coords.T
    z128 = jnp.zeros((NT, F), _f32)
    agg = _sc_agg(ef, rc2d, z128)
    csp = _sc_coords(phi_pack, rc2d, coords3, z128)

    wna = Wn[:, :F].T.astype(_bf16)
    wnb = Wn[:, F:].T.astype(_bf16)
    bn2 = bn.reshape(1, H)
    node_out, c_out = _node_mlp(node_feats, agg, co16, csp, wna, wnb, bn2)
    return node_out, ef, c_out[:, :3]


# speedup vs baseline: 6.2281x; 1.3344x over previous
import dataclasses

import jax
import jax.numpy as jnp
from jax import lax
from jax.experimental import pallas as pl
from jax.experimental.pallas import tpu as pltpu
from jax.experimental.pallas import tpu_sc as plsc

N = 10000
E = 320000
F = 128
H = 256
A = 17
TW = 144
EB = 2560
PB = EB // 128
NB = 1000

_bf16 = jnp.bfloat16
_f32 = jnp.float32


GW = 128


def _sc_gather(T, rc2d):
    mesh = plsc.VectorSubcoreMesh(core_axis_name="c", subcore_axis_name="s")

    @pl.kernel(out_type=jax.ShapeDtypeStruct((E, 2 * F), _f32), mesh=mesh)
    def k(t_hbm, i_hbm, o_hbm):
        def body(ir_vmem, ic_vmem, o1_vmem, o2_vmem):
            pltpu.sync_copy(t_hbm.at[ir_vmem.at[0]], o1_vmem)
            pltpu.sync_copy(t_hbm.at[ic_vmem.at[0]], o2_vmem)

        pltpu.emit_pipeline(
            body,
            grid=(E // GW,),
            in_specs=[pl.BlockSpec((1, GW), lambda i: (0, i)),
                      pl.BlockSpec((1, GW), lambda i: (0, E // GW + i))],
            out_specs=[pl.BlockSpec((GW, F), lambda i: (i, 0)),
                       pl.BlockSpec((GW, F), lambda i: (i, 1))],
            core_axis_name=("c", "s"),
            dimension_semantics=(pltpu.PARALLEL,),
        )(i_hbm, i_hbm, o_hbm, o_hbm)

    return k(T, rc2d)


EBC = 128
NT = N // 16


NW = 624


def _sc_agg(ef, rc2d, z128):
    mesh = plsc.VectorSubcoreMesh(core_axis_name="c", subcore_axis_name="s")

    @pl.kernel(
        out_type=jax.ShapeDtypeStruct((N, H), _f32),
        mesh=mesh,
        scratch_types=[pltpu.VMEM_SHARED((N, F), _f32)],
    )
    def k(ef_hbm, rc_hbm, z128_hbm, agg_hbm, agg_sh):
        cid = lax.axis_index("c")
        sid = lax.axis_index("s")
        pltpu.sync_copy(z128_hbm, agg_sh.at[pl.ds(sid * NT, NT)])
        plsc.subcore_barrier()

        def body(ef_vmem, ridx_vmem):
            pltpu.sync_copy(ef_vmem, agg_sh.at[ridx_vmem.at[0]], add=True)

        pltpu.emit_pipeline(
            body,
            grid=(E // EBC,),
            in_specs=[
                pl.BlockSpec((EBC, F), lambda i: (i, cid)),
                pl.BlockSpec((1, EBC), lambda i: (0, i)),
            ],
            core_axis_name="s",
            dimension_semantics=(pltpu.PARALLEL,),
        )(ef_hbm, rc_hbm)

        plsc.subcore_barrier()
        pltpu.sync_copy(agg_sh.at[pl.ds(sid * NW, NW)],
                        agg_hbm.at[pl.ds(sid * NW, NW), pl.ds(cid * F, F)])

        @pl.when(sid == 15)
        def _tail():
            pltpu.sync_copy(agg_sh.at[pl.ds(16 * NW, N - 16 * NW)],
                            agg_hbm.at[pl.ds(16 * NW, N - 16 * NW), pl.ds(cid * F, F)])

    return k(ef, rc2d, z128)


def _sc_coords(phi_pack, rc2d, coords3, z128):
    mesh = plsc.VectorSubcoreMesh(core_axis_name="c", subcore_axis_name="s")

    cp = pltpu.CompilerParams()
    if "needs_layout_passes" in pltpu.CompilerParams.__dataclass_fields__:
        cp = dataclasses.replace(cp, needs_layout_passes=False)

    @pl.kernel(
        out_type=jax.ShapeDtypeStruct((2, N, 128), _f32),
        mesh=mesh,
        compiler_params=cp,
        scratch_types=[
            pltpu.VMEM_SHARED((N, 128), _f32),
            pltpu.VMEM((3, N), _f32),
            pltpu.VMEM((64, 128), _f32),
            pltpu.VMEM((1, 64), jnp.int32),
        ],
    )
    def k(phi_hbm, rc_hbm, c3_hbm, z128_hbm, csp_hbm, cs_sh, ctab, stage, idx_s):
        cid = lax.axis_index("c")
        sid = lax.axis_index("s")
        pltpu.sync_copy(c3_hbm, ctab)
        pltpu.sync_copy(z128_hbm, cs_sh.at[pl.ds(sid * NT, NT)])
        pltpu.sync_copy(z128_hbm.at[pl.ds(0, 64)], stage)
        plsc.subcore_barrier()

        lanes = lax.iota(jnp.int32, 16)

        def body(ridx_vmem, cidx_vmem, phi_vmem):
            for j in range(4):
                sl = pl.ds(cid * 64 + j * 16, 16)
                idxr = ridx_vmem[0, sl]
                idxc = cidx_vmem[0, sl]
                p = phi_vmem[0, 0, sl]
                idx_s[0, pl.ds(j * 16, 16)] = idxr
                rows = jnp.full((16,), j * 16, jnp.int32) + lanes
                for d in range(3):
                    dsplat = jnp.full((16,), d, jnp.int32)
                    a = plsc.load_gather(ctab, [dsplat, idxr])
                    b = plsc.load_gather(ctab, [dsplat, idxc])
                    plsc.store_scatter(stage, [rows, dsplat], (a - b) * p)
            pltpu.sync_copy(stage, cs_sh.at[idx_s.at[0]], add=True)

        pltpu.emit_pipeline(
            body,
            grid=(E // EBC,),
            in_specs=[
                pl.BlockSpec((1, EBC), lambda i: (0, i)),
                pl.BlockSpec((1, EBC), lambda i: (0, E // EBC + i)),
                pl.BlockSpec((1, 1, 128), lambda i: (i // PB, i % PB, 0)),
            ],
            core_axis_name="s",
            dimension_semantics=(pltpu.PARALLEL,),
        )(rc_hbm, rc_hbm, phi_hbm)

        plsc.subcore_barrier()
        pltpu.sync_copy(cs_sh.at[pl.ds(sid * NW, NW)],
                        csp_hbm.at[cid].at[pl.ds(sid * NW, NW)])

        @pl.when(sid == 15)
        def _tail():
            pltpu.sync_copy(cs_sh.at[pl.ds(16 * NW, N - 16 * NW)],
                            csp_hbm.at[cid].at[pl.ds(16 * NW, N - 16 * NW)])

    return k(phi_pack, rc2d, coords3, z128)


def _edge_body(gs, ea, ws, wa, be, wc1, bc1, wc2, ef_out, phi_out):
    g = gs[...].astype(_bf16)
    x = jnp.dot(g, ws[...], preferred_element_type=_f32)
    x = x + jnp.dot(ea[...].astype(_bf16), wa[...], preferred_element_type=_f32)
    x = x + be[...]
    ef = jnp.maximum(x, 0.0)
    ef_out[...] = ef
    h = jnp.dot(ef.astype(_bf16), wc1[...], preferred_element_type=_f32) + bc1[...]
    h = jnp.maximum(h, 0.0)
    phi = jnp.dot(h.astype(_bf16), wc2[...], preferred_element_type=_f32)
    phi_out[...] = phi.reshape(1, EB // 128, 128)


def _edge_mlp(G, edge_attr, ws, wa, be2, wc1b, bc12, wc22):
    nsteps = E // EB
    return pl.pallas_call(
        _edge_body,
        grid=(nsteps,),
        in_specs=[
            pl.BlockSpec((EB, 2 * F), lambda i: (i, 0)),
            pl.BlockSpec((EB, A), lambda i: (i, 0)),
            pl.BlockSpec((2 * F, H), lambda i: (0, 0)),
            pl.BlockSpec((A, H), lambda i: (0, 0)),
            pl.BlockSpec((1, H), lambda i: (0, 0)),
            pl.BlockSpec((H, H), lambda i: (0, 0)),
            pl.BlockSpec((1, H), lambda i: (0, 0)),
            pl.BlockSpec((H, 1), lambda i: (0, 0)),
        ],
        out_specs=[
            pl.BlockSpec((EB, H), lambda i: (i, 0)),
            pl.BlockSpec((1, EB // 128, 128), lambda i: (i, 0, 0)),
        ],
        out_shape=[
            jax.ShapeDtypeStruct((E, H), _f32),
            jax.ShapeDtypeStruct((E // EB, EB // 128, 128), _f32),
        ],
    )(G, edge_attr, ws, wa, be2, wc1b, bc12, wc22)


def _node_body(nf, agg, co, cs0, cs1, wna, wnb, bn, no_out, c_out):
    x = jnp.dot(nf[...].astype(_bf16), wna[...], preferred_element_type=_f32)
    x = x + jnp.dot(agg[...].astype(_bf16), wnb[...], preferred_element_type=_f32)
    x = x + bn[...]
    no_out[...] = jnp.maximum(x, 0.0)
    c_out[...] = co[...] + cs0[0][:, :16] + cs1[0][:, :16]


def _node_mlp(node_feats, agg, co16, csp, wna, wnb, bn2):
    return pl.pallas_call(
        _node_body,
        grid=(N // NB,),
        in_specs=[
            pl.BlockSpec((NB, F), lambda i: (i, 0)),
            pl.BlockSpec((NB, H), lambda i: (i, 0)),
            pl.BlockSpec((NB, 16), lambda i: (i, 0)),
            pl.BlockSpec((1, NB, 128), lambda i: (0, i, 0)),
            pl.BlockSpec((1, NB, 128), lambda i: (1, i, 0)),
            pl.BlockSpec((F, H), lambda i: (0, 0)),
            pl.BlockSpec((H, H), lambda i: (0, 0)),
            pl.BlockSpec((1, H), lambda i: (0, 0)),
        ],
        out_specs=[
            pl.BlockSpec((NB, H), lambda i: (i, 0)),
            pl.BlockSpec((NB, 16), lambda i: (i, 0)),
        ],
        out_shape=[
            jax.ShapeDtypeStruct((N, H), _f32),
            jax.ShapeDtypeStruct((N, 16), _f32),
        ],
    )(node_feats, agg, co16, csp, csp, wna, wnb, bn2)


def kernel(node_feats, edge_index, edge_attr, coords, We, be, Wn, bn, Wc1, bc1, Wc2):
    co16 = jnp.pad(coords, ((0, 0), (0, 13)))
    rc2d = edge_index.reshape(1, 2 * E)

    G = _sc_gather(node_feats, rc2d)

    ws = We[:, :2 * F].T.astype(_bf16)
    wa = We[:, 2 * F:].T.astype(_bf16)
    be2 = be.reshape(1, H)
    wc1b = Wc1.T.astype(_bf16)
    bc12 = bc1.reshape(1, H)
    wc22 = Wc2.reshape(H, 1).astype(_bf16)

    ef, phi_pack = _edge_mlp(G, edge_attr, ws, wa, be2, wc1b, bc12, wc22)

    coords3 = coords.T
    z128 = jnp.zeros((NT, F), _f32)
    agg = _sc_agg(ef, rc2d, z128)
    csp = _sc_coords(phi_pack, rc2d, coords3, z128)

    wna = Wn[:, :F].T.astype(_bf16)
    wnb = Wn[:, F:].T.astype(_bf16)
    bn2 = bn.reshape(1, H)
    node_out, c_out = _node_mlp(node_feats, agg, co16, csp, wna, wnb, bn2)
    return node_out, ef, c_out[:, :3]


# speedup vs baseline: 6.3059x; 1.0125x over previous
import dataclasses

import jax
import jax.numpy as jnp
from jax import lax
from jax.experimental import pallas as pl
from jax.experimental.pallas import tpu as pltpu
from jax.experimental.pallas import tpu_sc as plsc

N = 10000
E = 320000
F = 128
H = 256
A = 17
TW = 144
EB = 3200
PB = EB // 128
NB = 1000

_bf16 = jnp.bfloat16
_f32 = jnp.float32


GW = 128


def _sc_gather(T, rc2d):
    mesh = plsc.VectorSubcoreMesh(core_axis_name="c", subcore_axis_name="s")

    @pl.kernel(out_type=jax.ShapeDtypeStruct((E, 2 * F), _f32), mesh=mesh)
    def k(t_hbm, i_hbm, o_hbm):
        def body(ir_vmem, ic_vmem, o1_vmem, o2_vmem):
            pltpu.sync_copy(t_hbm.at[ir_vmem.at[0]], o1_vmem)
            pltpu.sync_copy(t_hbm.at[ic_vmem.at[0]], o2_vmem)

        pltpu.emit_pipeline(
            body,
            grid=(E // GW,),
            in_specs=[pl.BlockSpec((1, GW), lambda i: (0, i)),
                      pl.BlockSpec((1, GW), lambda i: (0, E // GW + i))],
            out_specs=[pl.BlockSpec((GW, F), lambda i: (i, 0)),
                       pl.BlockSpec((GW, F), lambda i: (i, 1))],
            core_axis_name=("c", "s"),
            dimension_semantics=(pltpu.PARALLEL,),
        )(i_hbm, i_hbm, o_hbm, o_hbm)

    return k(T, rc2d)


EBC = 128
NT = N // 16


NW = 624


def _sc_agg(ef, rc2d, z128):
    mesh = plsc.VectorSubcoreMesh(core_axis_name="c", subcore_axis_name="s")

    @pl.kernel(
        out_type=jax.ShapeDtypeStruct((N, H), _f32),
        mesh=mesh,
        scratch_types=[pltpu.VMEM_SHARED((N, F), _f32)],
    )
    def k(ef_hbm, rc_hbm, z128_hbm, agg_hbm, agg_sh):
        cid = lax.axis_index("c")
        sid = lax.axis_index("s")
        pltpu.sync_copy(z128_hbm, agg_sh.at[pl.ds(sid * NT, NT)])
        plsc.subcore_barrier()

        def body(ef_vmem, ridx_vmem):
            pltpu.sync_copy(ef_vmem, agg_sh.at[ridx_vmem.at[0]], add=True)

        pltpu.emit_pipeline(
            body,
            grid=(E // EBC,),
            in_specs=[
                pl.BlockSpec((EBC, F), lambda i: (i, cid)),
                pl.BlockSpec((1, EBC), lambda i: (0, i)),
            ],
            core_axis_name="s",
            dimension_semantics=(pltpu.PARALLEL,),
        )(ef_hbm, rc_hbm)

        plsc.subcore_barrier()
        pltpu.sync_copy(agg_sh.at[pl.ds(sid * NW, NW)],
                        agg_hbm.at[pl.ds(sid * NW, NW), pl.ds(cid * F, F)])

        @pl.when(sid == 15)
        def _tail():
            pltpu.sync_copy(agg_sh.at[pl.ds(16 * NW, N - 16 * NW)],
                            agg_hbm.at[pl.ds(16 * NW, N - 16 * NW), pl.ds(cid * F, F)])

    return k(ef, rc2d, z128)


def _sc_coords(phi_pack, rc2d, coords3, z128):
    mesh = plsc.VectorSubcoreMesh(core_axis_name="c", subcore_axis_name="s")

    cp = pltpu.CompilerParams()
    if "needs_layout_passes" in pltpu.CompilerParams.__dataclass_fields__:
        cp = dataclasses.replace(cp, needs_layout_passes=False)

    @pl.kernel(
        out_type=jax.ShapeDtypeStruct((2, N, 128), _f32),
        mesh=mesh,
        compiler_params=cp,
        scratch_types=[
            pltpu.VMEM_SHARED((N, 128), _f32),
            pltpu.VMEM((3, N), _f32),
            pltpu.VMEM((64, 128), _f32),
            pltpu.VMEM((1, 64), jnp.int32),
        ],
    )
    def k(phi_hbm, rc_hbm, c3_hbm, z128_hbm, csp_hbm, cs_sh, ctab, stage, idx_s):
        cid = lax.axis_index("c")
        sid = lax.axis_index("s")
        pltpu.sync_copy(c3_hbm, ctab)
        pltpu.sync_copy(z128_hbm, cs_sh.at[pl.ds(sid * NT, NT)])
        pltpu.sync_copy(z128_hbm.at[pl.ds(0, 64)], stage)
        plsc.subcore_barrier()

        lanes = lax.iota(jnp.int32, 16)

        def body(ridx_vmem, cidx_vmem, phi_vmem):
            for j in range(4):
                sl = pl.ds(cid * 64 + j * 16, 16)
                idxr = ridx_vmem[0, sl]
                idxc = cidx_vmem[0, sl]
                p = phi_vmem[0, 0, sl]
                idx_s[0, pl.ds(j * 16, 16)] = idxr
                rows = jnp.full((16,), j * 16, jnp.int32) + lanes
                for d in range(3):
                    dsplat = jnp.full((16,), d, jnp.int32)
                    a = plsc.load_gather(ctab, [dsplat, idxr])
                    b = plsc.load_gather(ctab, [dsplat, idxc])
                    plsc.store_scatter(stage, [rows, dsplat], (a - b) * p)
            pltpu.sync_copy(stage, cs_sh.at[idx_s.at[0]], add=True)

        pltpu.emit_pipeline(
            body,
            grid=(E // EBC,),
            in_specs=[
                pl.BlockSpec((1, EBC), lambda i: (0, i)),
                pl.BlockSpec((1, EBC), lambda i: (0, E // EBC + i)),
                pl.BlockSpec((1, 1, 128), lambda i: (i // PB, i % PB, 0)),
            ],
            core_axis_name="s",
            dimension_semantics=(pltpu.PARALLEL,),
        )(rc_hbm, rc_hbm, phi_hbm)

        plsc.subcore_barrier()
        pltpu.sync_copy(cs_sh.at[pl.ds(sid * NW, NW)],
                        csp_hbm.at[cid].at[pl.ds(sid * NW, NW)])

        @pl.when(sid == 15)
        def _tail():
            pltpu.sync_copy(cs_sh.at[pl.ds(16 * NW, N - 16 * NW)],
                            csp_hbm.at[cid].at[pl.ds(16 * NW, N - 16 * NW)])

    return k(phi_pack, rc2d, coords3, z128)


def _edge_body(gs, ea, ws, wa, be, wc1, bc1, wc2, ef_out, phi_out):
    g = gs[...].astype(_bf16)
    x = jnp.dot(g, ws[...], preferred_element_type=_f32)
    x = x + jnp.dot(ea[...].astype(_bf16), wa[...], preferred_element_type=_f32)
    x = x + be[...]
    ef = jnp.maximum(x, 0.0)
    ef_out[...] = ef
    h = jnp.dot(ef.astype(_bf16), wc1[...], preferred_element_type=_f32) + bc1[...]
    h = jnp.maximum(h, 0.0)
    phi = jnp.dot(h.astype(_bf16), wc2[...], preferred_element_type=_f32)
    phi_out[...] = phi.reshape(1, EB // 128, 128)


def _edge_mlp(G, edge_attr, ws, wa, be2, wc1b, bc12, wc22):
    nsteps = E // EB
    return pl.pallas_call(
        _edge_body,
        grid=(nsteps,),
        in_specs=[
            pl.BlockSpec((EB, 2 * F), lambda i: (i, 0)),
            pl.BlockSpec((EB, A), lambda i: (i, 0)),
            pl.BlockSpec((2 * F, H), lambda i: (0, 0)),
            pl.BlockSpec((A, H), lambda i: (0, 0)),
            pl.BlockSpec((1, H), lambda i: (0, 0)),
            pl.BlockSpec((H, H), lambda i: (0, 0)),
            pl.BlockSpec((1, H), lambda i: (0, 0)),
            pl.BlockSpec((H, 1), lambda i: (0, 0)),
        ],
        out_specs=[
            pl.BlockSpec((EB, H), lambda i: (i, 0)),
            pl.BlockSpec((1, EB // 128, 128), lambda i: (i, 0, 0)),
        ],
        out_shape=[
            jax.ShapeDtypeStruct((E, H), _f32),
            jax.ShapeDtypeStruct((E // EB, EB // 128, 128), _f32),
        ],
    )(G, edge_attr, ws, wa, be2, wc1b, bc12, wc22)


def _node_body(nf, agg, co, cs0, cs1, wna, wnb, bn, no_out, c_out):
    x = jnp.dot(nf[...].astype(_bf16), wna[...], preferred_element_type=_f32)
    x = x + jnp.dot(agg[...].astype(_bf16), wnb[...], preferred_element_type=_f32)
    x = x + bn[...]
    no_out[...] = jnp.maximum(x, 0.0)
    c_out[...] = co[...] + cs0[0][:, :16] + cs1[0][:, :16]


def _node_mlp(node_feats, agg, co16, csp, wna, wnb, bn2):
    return pl.pallas_call(
        _node_body,
        grid=(N // NB,),
        in_specs=[
            pl.BlockSpec((NB, F), lambda i: (i, 0)),
            pl.BlockSpec((NB, H), lambda i: (i, 0)),
            pl.BlockSpec((NB, 16), lambda i: (i, 0)),
            pl.BlockSpec((1, NB, 128), lambda i: (0, i, 0)),
            pl.BlockSpec((1, NB, 128), lambda i: (1, i, 0)),
            pl.BlockSpec((F, H), lambda i: (0, 0)),
            pl.BlockSpec((H, H), lambda i: (0, 0)),
            pl.BlockSpec((1, H), lambda i: (0, 0)),
        ],
        out_specs=[
            pl.BlockSpec((NB, H), lambda i: (i, 0)),
            pl.BlockSpec((NB, 16), lambda i: (i, 0)),
        ],
        out_shape=[
            jax.ShapeDtypeStruct((N, H), _f32),
            jax.ShapeDtypeStruct((N, 16), _f32),
        ],
    )(node_feats, agg, co16, csp, csp, wna, wnb, bn2)


def kernel(node_feats, edge_index, edge_attr, coords, We, be, Wn, bn, Wc1, bc1, Wc2):
    co16 = jnp.pad(coords, ((0, 0), (0, 13)))
    rc2d = edge_index.reshape(1, 2 * E)

    G = _sc_gather(node_feats, rc2d)

    ws = We[:, :2 * F].T.astype(_bf16)
    wa = We[:, 2 * F:].T.astype(_bf16)
    be2 = be.reshape(1, H)
    wc1b = Wc1.T.astype(_bf16)
    bc12 = bc1.reshape(1, H)
    wc22 = Wc2.reshape(H, 1).astype(_bf16)

    ef, phi_pack = _edge_mlp(G, edge_attr, ws, wa, be2, wc1b, bc12, wc22)

    coords3 = coords.T
    z128 = jnp.zeros((NT, F), _f32)
    agg = _sc_agg(ef, rc2d, z128)
    csp = _sc_coords(phi_pack, rc2d, coords3, z128)

    wna = Wn[:, :F].T.astype(_bf16)
    wnb = Wn[:, F:].T.astype(_bf16)
    bn2 = bn.reshape(1, H)
    node_out, c_out = _node_mlp(node_feats, agg, co16, csp, wna, wnb, bn2)
    return node_out, ef, c_out[:, :3]
